# Initial kernel scaffold; baseline (speedup 1.0000x reference)
import jax
import jax.numpy as jnp
from jax.experimental import pallas as pl


def kernel(x, w1, b1, w2, b2):
    raise NotImplementedError("write your pallas kernel here")


# speedup vs baseline: 1.2434x; 1.2434x over previous
import functools

import jax
import jax.numpy as jnp
from jax.experimental import pallas as pl
from jax.experimental.pallas import tpu as pltpu

_SQRT_2_OVER_PI = 0.7978845608028654

_RT = 16
_BM = 4


def _gelu_tanh(x):
    inner = _SQRT_2_OVER_PI * (x + 0.044715 * (x * x * x))
    return 0.5 * x * (1.0 + jnp.tanh(inner))


def _mlp_kernel(x_ref, w1_ref, b1_ref, w2_ref, b2_ref, o_ref, *,
                bm, rt, cin, h1, co, q, s):
    w1 = w1_ref[...]
    b1 = b1_ref[...]
    w2 = w2_ref[...]
    b2 = b2_ref[...]
    cr = rt * cin
    for c in range(bm):
        xc = x_ref[c * cr:(c + 1) * cr, :].astype(jnp.bfloat16)
        h = jnp.dot(w1, xc, preferred_element_type=jnp.float32) + b1
        g = _gelu_tanh(h).astype(jnp.bfloat16)
        o = jnp.dot(w2, g, preferred_element_type=jnp.float32) + b2
        o4 = o.reshape(rt, co, s, q).transpose(0, 2, 1, 3)
        o_ref[c * rt:(c + 1) * rt] = o4.reshape(rt, s * co, q)


def kernel(x, w1, b1, w2, b2):
    B, A, P, Q = x.shape
    h1, cin = w1.shape
    s = P // cin
    co = w2.shape[0]
    assert P == cin * s and s * Q % 128 == 0
    R = B * A
    rt, bm = _RT, _BM
    rows_per_step = bm * rt
    assert R % rows_per_step == 0

    x2 = x.reshape(R * cin, s * Q)

    eye = jnp.eye(rt, dtype=jnp.float32)
    w1bd = (eye[:, None, :, None] * w1[None, :, None, :]).reshape(
        rt * h1, rt * cin).astype(jnp.bfloat16)
    w2bd = (eye[:, None, :, None] * w2[None, :, None, :]).reshape(
        rt * co, rt * h1).astype(jnp.bfloat16)
    b1bd = jnp.tile(b1, rt).reshape(rt * h1, 1).astype(jnp.float32)
    b2bd = jnp.tile(b2, rt).reshape(rt * co, 1).astype(jnp.float32)

    kfn = functools.partial(_mlp_kernel, bm=bm, rt=rt, cin=cin, h1=h1,
                            co=co, q=Q, s=s)
    flops = int(2 * R * Q * s * (h1 * cin + co * h1))
    cost = pl.CostEstimate(
        flops=flops,
        transcendentals=int(R * Q * s * h1),
        bytes_accessed=int(x.size * 4 + R * s * co * Q * 4))

    y = pl.pallas_call(
        kfn,
        out_shape=jax.ShapeDtypeStruct((R, s * co, Q), x.dtype),
        grid=(R // rows_per_step,),
        in_specs=[
            pl.BlockSpec((rows_per_step * cin, s * Q), lambda i: (i, 0)),
            pl.BlockSpec((rt * h1, rt * cin), lambda i: (0, 0)),
            pl.BlockSpec((rt * h1, 1), lambda i: (0, 0)),
            pl.BlockSpec((rt * co, rt * h1), lambda i: (0, 0)),
            pl.BlockSpec((rt * co, 1), lambda i: (0, 0)),
        ],
        out_specs=pl.BlockSpec((rows_per_step, s * co, Q),
                               lambda i: (i, 0, 0)),
        compiler_params=pltpu.CompilerParams(
            dimension_semantics=("parallel",),
            vmem_limit_bytes=32 * 1024 * 1024),
        cost_estimate=cost,
    )(x2, w1bd, b1bd, w2bd, b2bd)
    return y.reshape(B, A, s * co, Q)


# speedup vs baseline: 1.4813x; 1.1913x over previous
import functools

import jax
import jax.numpy as jnp
from jax.experimental import pallas as pl
from jax.experimental.pallas import tpu as pltpu

_SQRT_2_OVER_PI = 0.7978845608028654

_RT = 16
_BM = 8


def _mlp_kernel(x_ref, w1_ref, b1_ref, w2_ref, b2_ref, o_ref, *,
                bm, rt, cin, h1, co, q, s):
    w1 = w1_ref[...]
    b1 = b1_ref[...]
    w2 = w2_ref[...]
    b2 = b2_ref[...]
    cr = rt * cin
    c1 = jnp.bfloat16(_SQRT_2_OVER_PI)
    c2 = jnp.bfloat16(0.044715 * _SQRT_2_OVER_PI)
    half = jnp.bfloat16(0.5)
    for c in range(bm):
        xc = x_ref[c * cr:(c + 1) * cr, :].astype(jnp.bfloat16)
        h = (jnp.dot(w1, xc, preferred_element_type=jnp.float32)
             + b1).astype(jnp.bfloat16)
        h2 = h * h
        t = jnp.tanh(h * (c1 + c2 * h2))
        u = half * h
        g = u + u * t
        o = jnp.dot(w2, g, preferred_element_type=jnp.float32) + b2
        o3 = o.reshape(rt, co, s * q)
        for i in range(s):
            o_ref[c * rt:(c + 1) * rt, i * co:(i + 1) * co, :] = (
                o3[:, :, i * q:(i + 1) * q])


def kernel(x, w1, b1, w2, b2):
    B, A, P, Q = x.shape
    h1, cin = w1.shape
    s = P // cin
    co = w2.shape[0]
    assert P == cin * s and s * Q % 128 == 0
    R = B * A
    rt, bm = _RT, _BM
    rows_per_step = bm * rt
    assert R % rows_per_step == 0

    x2 = x.reshape(R * cin, s * Q)

    eye = jnp.eye(rt, dtype=jnp.float32)
    w1bd = (eye[:, None, :, None] * w1[None, :, None, :]).reshape(
        rt * h1, rt * cin).astype(jnp.bfloat16)
    w2bd = (eye[:, None, :, None] * w2[None, :, None, :]).reshape(
        rt * co, rt * h1).astype(jnp.bfloat16)
    b1bd = jnp.tile(b1, rt).reshape(rt * h1, 1).astype(jnp.float32)
    b2bd = jnp.tile(b2, rt).reshape(rt * co, 1).astype(jnp.float32)

    kfn = functools.partial(_mlp_kernel, bm=bm, rt=rt, cin=cin, h1=h1,
                            co=co, q=Q, s=s)
    flops = int(2 * R * Q * s * (h1 * cin + co * h1))
    cost = pl.CostEstimate(
        flops=flops,
        transcendentals=int(R * Q * s * h1),
        bytes_accessed=int(x.size * 4 + R * s * co * Q * 4))

    y = pl.pallas_call(
        kfn,
        out_shape=jax.ShapeDtypeStruct((R, s * co, Q), x.dtype),
        grid=(R // rows_per_step,),
        in_specs=[
            pl.BlockSpec((rows_per_step * cin, s * Q), lambda i: (i, 0)),
            pl.BlockSpec((rt * h1, rt * cin), lambda i: (0, 0)),
            pl.BlockSpec((rt * h1, 1), lambda i: (0, 0)),
            pl.BlockSpec((rt * co, rt * h1), lambda i: (0, 0)),
            pl.BlockSpec((rt * co, 1), lambda i: (0, 0)),
        ],
        out_specs=pl.BlockSpec((rows_per_step, s * co, Q),
                               lambda i: (i, 0, 0)),
        compiler_params=pltpu.CompilerParams(
            dimension_semantics=("parallel",),
            vmem_limit_bytes=32 * 1024 * 1024),
        cost_estimate=cost,
    )(x2, w1bd, b1bd, w2bd, b2bd)
    return y.reshape(B, A, s * co, Q)


# speedup vs baseline: 2.2965x; 1.5503x over previous
import functools

import jax
import jax.numpy as jnp
from jax.experimental import pallas as pl
from jax.experimental.pallas import tpu as pltpu

_SQRT_2_OVER_PI = 0.7978845608028654

_RT = 16
_BM = 16


def _mlp_kernel(x_ref, w1_ref, b1_ref, w2_ref, b2_ref, o_ref, *,
                bm, rt, cin, h1, co, q, s):
    w1 = w1_ref[...]
    b1 = b1_ref[...]
    w2 = w2_ref[...]
    b2 = b2_ref[...]
    cr = rt * cin
    c1 = jnp.bfloat16(_SQRT_2_OVER_PI)
    c2 = jnp.bfloat16(0.044715 * _SQRT_2_OVER_PI)
    half = jnp.bfloat16(0.5)
    hs = []
    for c in range(bm):
        xc = x_ref[c * cr:(c + 1) * cr, :].astype(jnp.bfloat16)
        hs.append(jnp.dot(w1, xc, preferred_element_type=jnp.float32)
                  .astype(jnp.bfloat16) + b1)
    gs = []
    for c in range(bm):
        h = hs[c]
        h2 = h * h
        t = jnp.tanh(h * (c1 + c2 * h2))
        u = half * h
        gs.append(u + u * t)
    for c in range(bm):
        o = jnp.dot(w2, gs[c], preferred_element_type=jnp.float32) + b2
        o3 = o.reshape(rt, co, s * q)
        for i in range(s):
            o_ref[c * rt:(c + 1) * rt, i * co:(i + 1) * co, :] = (
                o3[:, :, i * q:(i + 1) * q])


def kernel(x, w1, b1, w2, b2):
    B, A, P, Q = x.shape
    h1, cin = w1.shape
    s = P // cin
    co = w2.shape[0]
    assert P == cin * s and s * Q % 128 == 0
    R = B * A
    rt, bm = _RT, _BM
    rows_per_step = bm * rt
    assert R % rows_per_step == 0

    x2 = x.reshape(R * cin, s * Q)

    eye = jnp.eye(rt, dtype=jnp.float32)
    w1bd = (eye[:, None, :, None] * w1[None, :, None, :]).reshape(
        rt * h1, rt * cin).astype(jnp.bfloat16)
    w2bd = (eye[:, None, :, None] * w2[None, :, None, :]).reshape(
        rt * co, rt * h1).astype(jnp.bfloat16)
    b1bd = jnp.tile(b1, rt).reshape(rt * h1, 1).astype(jnp.bfloat16)
    b2bd = jnp.tile(b2, rt).reshape(rt * co, 1).astype(jnp.float32)

    kfn = functools.partial(_mlp_kernel, bm=bm, rt=rt, cin=cin, h1=h1,
                            co=co, q=Q, s=s)
    flops = int(2 * R * Q * s * (h1 * cin + co * h1))
    cost = pl.CostEstimate(
        flops=flops,
        transcendentals=int(R * Q * s * h1),
        bytes_accessed=int(x.size * 4 + R * s * co * Q * 4))

    y = pl.pallas_call(
        kfn,
        out_shape=jax.ShapeDtypeStruct((R, s * co, Q), x.dtype),
        grid=(R // rows_per_step,),
        in_specs=[
            pl.BlockSpec((rows_per_step * cin, s * Q), lambda i: (i, 0)),
            pl.BlockSpec((rt * h1, rt * cin), lambda i: (0, 0)),
            pl.BlockSpec((rt * h1, 1), lambda i: (0, 0)),
            pl.BlockSpec((rt * co, rt * h1), lambda i: (0, 0)),
            pl.BlockSpec((rt * co, 1), lambda i: (0, 0)),
        ],
        out_specs=pl.BlockSpec((rows_per_step, s * co, Q),
                               lambda i: (i, 0, 0)),
        compiler_params=pltpu.CompilerParams(
            dimension_semantics=("parallel",),
            vmem_limit_bytes=32 * 1024 * 1024),
        cost_estimate=cost,
    )(x2, w1bd, b1bd, w2bd, b2bd)
    return y.reshape(B, A, s * co, Q)
